# baseline (speedup 1.0000x reference)
import functools

import jax
import jax.numpy as jnp
from jax import lax
from jax.experimental import pallas as pl
from jax.experimental.pallas import tpu as pltpu
from jax.experimental.pallas import tpu_sc as plsc

N_CLS = 1000
N_CTX = 16
D = 512
SEQ = 77
SUF = SEQ - 1 - N_CTX
B = 1024
W_PRE = D
W_CTX = N_CTX * D
W_SUF = SUF * D
W_ROW = SEQ * D
SEQ_PAD = 80

NC, NS = 2, 16
NW = NC * NS
BPW = B // NW
CTX_CH = 4
SUF_CH = 2


def _sc_body(cls_w, cls_c, cls_s,
             pre_n, ctx_n, suf_n, pre_p, ctx_p, suf_p,
             tok_n, tok_p,
             out, out_tok,
             idx_v, idxc_v, idxs_v, pbuf, cbuf, sbuf, tbuf, sem):
    wid = lax.axis_index("s") * NC + lax.axis_index("c")
    base = wid * BPW

    pltpu.sync_copy(cls_w.at[wid], idx_v)
    pltpu.sync_copy(cls_c.at[wid], idxc_v)
    pltpu.sync_copy(cls_s.at[wid], idxs_v)

    for pre_t, ctx_t, suf_t, tok_t, row0 in (
        (pre_n, ctx_n, suf_n, tok_n, 0),
        (pre_p, ctx_p, suf_p, tok_p, B),
    ):
        ob = row0 + base
        pltpu.async_copy(pre_t.at[idx_v], pbuf, sem).wait()
        pltpu.sync_copy(pbuf, out.at[pl.ds(ob, BPW), pl.ds(0, W_PRE)])
        pltpu.async_copy(tok_t.at[idx_v], tbuf, sem).wait()
        pltpu.sync_copy(tbuf, out_tok.at[pl.ds(ob, BPW), :])
        for j in range(BPW // CTX_CH):
            pltpu.async_copy(ctx_t.at[idxc_v.at[j]], cbuf, sem).wait()
            pltpu.sync_copy(
                cbuf, out.at[pl.ds(ob + j * CTX_CH, CTX_CH),
                             pl.ds(W_PRE, W_CTX)])
        for j in range(BPW // SUF_CH):
            pltpu.async_copy(suf_t.at[idxs_v.at[j]], sbuf, sem).wait()
            pltpu.sync_copy(
                sbuf, out.at[pl.ds(ob + j * SUF_CH, SUF_CH),
                             pl.ds(W_PRE + W_CTX, W_SUF)])


@jax.jit
def _prompt_gather(cls_id, ctx_pos2, ctx_neg2, pre_pos2, suf_pos2,
                   pre_neg2, suf_neg2, tok_neg, tok_pos):
    cls_w = cls_id.reshape(NW, BPW)
    cls_c = cls_id.reshape(NW, BPW // CTX_CH, CTX_CH)
    cls_s = cls_id.reshape(NW, BPW // SUF_CH, SUF_CH)
    mesh = plsc.VectorSubcoreMesh(core_axis_name="c", subcore_axis_name="s",
                                  num_cores=NC, num_subcores=NS)
    call = pl.kernel(
        _sc_body,
        out_type=(
            jax.ShapeDtypeStruct((2 * B, W_ROW), jnp.float32),
            jax.ShapeDtypeStruct((2 * B, SEQ_PAD), jnp.int32),
        ),
        mesh=mesh,
        scratch_types=[
            pltpu.VMEM((BPW,), jnp.int32),
            pltpu.VMEM((BPW // CTX_CH, CTX_CH), jnp.int32),
            pltpu.VMEM((BPW // SUF_CH, SUF_CH), jnp.int32),
            pltpu.VMEM((BPW, W_PRE), jnp.float32),
            pltpu.VMEM((CTX_CH, W_CTX), jnp.float32),
            pltpu.VMEM((SUF_CH, W_SUF), jnp.float32),
            pltpu.VMEM((BPW, SEQ_PAD), jnp.int32),
            pltpu.SemaphoreType.DMA,
        ],
        compiler_params=pltpu.CompilerParams(use_tc_tiling_on_sc=False),
    )
    return call(cls_w, cls_c, cls_s,
                pre_neg2, ctx_neg2, suf_neg2,
                pre_pos2, ctx_pos2, suf_pos2,
                tok_neg, tok_pos)


def kernel(cls_id, ctx_pos, ctx_neg, token_prefix_pos, token_suffix_pos,
           token_prefix_neg, token_suffix_neg, tokenized_prompts):
    n_cls = ctx_pos.shape[0]
    prompts2, tokenized = _prompt_gather(
        cls_id,
        ctx_pos.reshape(n_cls, W_CTX),
        ctx_neg.reshape(n_cls, W_CTX),
        token_prefix_pos.reshape(n_cls, W_PRE),
        token_suffix_pos.reshape(n_cls, W_SUF),
        token_prefix_neg.reshape(n_cls, W_PRE),
        token_suffix_neg.reshape(n_cls, W_SUF),
        jnp.pad(tokenized_prompts[:n_cls], ((0, 0), (0, SEQ_PAD - SEQ))),
        jnp.pad(tokenized_prompts[n_cls:], ((0, 0), (0, SEQ_PAD - SEQ))),
    )
    return prompts2.reshape(2 * B, SEQ, D), tokenized[:, :SEQ]


# speedup vs baseline: 1.0092x; 1.0092x over previous
import jax
import jax.numpy as jnp
from jax import lax
from jax.experimental import pallas as pl
from jax.experimental.pallas import tpu as pltpu
from jax.experimental.pallas import tpu_sc as plsc

N_CLS = 1000
N_CTX = 16
D = 512
SEQ = 77
SUF = SEQ - 1 - N_CTX
B = 1024
W_PRE = D
W_CTX = N_CTX * D
W_SUF = SUF * D
W_ROW = SEQ * D
SEQ_PAD = 80

NC, NS = 2, 16
NW = NC * NS
BPW = B // NW
CTX_CH = 2
SUF_CH = 1


def _sc_body(cls_w, cls_c, cls_s,
             pre_n, ctx_n, suf_n, pre_p, ctx_p, suf_p,
             tok_n, tok_p,
             out, out_tok,
             idx_v, idxc_v, idxs_v,
             pbuf, tbuf, cbuf0, cbuf1, sbuf0, sbuf1,
             gsem_p, ssem_p, gsem_t, ssem_t,
             gsem_c0, gsem_c1, ssem_c0, ssem_c1,
             gsem_s0, gsem_s1, ssem_s0, ssem_s1):
    wid = lax.axis_index("s") * NC + lax.axis_index("c")
    base = wid * BPW

    pltpu.sync_copy(cls_w.at[wid], idx_v)
    pltpu.sync_copy(cls_c.at[wid], idxc_v)
    pltpu.sync_copy(cls_s.at[wid], idxs_v)

    col_ctx, col_suf = W_PRE, W_PRE + W_CTX
    slots = {
        "s0": (sbuf0, gsem_s0, ssem_s0),
        "s1": (sbuf1, gsem_s1, ssem_s1),
        "c0": (cbuf0, gsem_c0, ssem_c0),
        "c1": (cbuf1, gsem_c1, ssem_c1),
        "p": (pbuf, gsem_p, ssem_p),
        "t": (tbuf, gsem_t, ssem_t),
    }

    task_list = []

    def add_suf(pol, tab, ob):
        for j in range(BPW // SUF_CH):
            slot = "s0" if j % 2 == 0 else "s1"
            idx = idxs_v.at[pol * (BPW // SUF_CH) + j]
            dst = out.at[pl.ds(ob + j * SUF_CH, SUF_CH), pl.ds(col_suf, W_SUF)]
            task_list.append((tab.at[idx], dst, slot))

    def add_ctx(pol, tab, ob):
        for j in range(BPW // CTX_CH):
            slot = "c0" if j % 2 == 0 else "c1"
            idx = idxc_v.at[pol * (BPW // CTX_CH) + j]
            dst = out.at[pl.ds(ob + j * CTX_CH, CTX_CH), pl.ds(col_ctx, W_CTX)]
            task_list.append((tab.at[idx], dst, slot))

    halves = ((pre_n, ctx_n, suf_n, tok_n, base),
              (pre_p, ctx_p, suf_p, tok_p, B + base))
    for pol, (pre_t, ctx_t, suf_t, tok_t, ob) in enumerate(halves):
        task_list.append((pre_t.at[idx_v],
                          out.at[pl.ds(ob, BPW), pl.ds(0, W_PRE)], "p"))
        task_list.append((tok_t.at[idx_v],
                          out_tok.at[pl.ds(ob, BPW), :], "t"))
        add_ctx(pol, ctx_t, ob)
        add_suf(pol, suf_t, ob)

    last_scatter = {}
    prev = None
    for src, dst, slot in task_list:
        buf, gsem, ssem = slots[slot]
        if slot in last_scatter:
            last_scatter.pop(slot).wait()
        g = pltpu.async_copy(src, buf, gsem)
        if prev is not None:
            pg, pdst, pslot = prev
            pbuf_, _, pssem = slots[pslot]
            pg.wait()
            last_scatter[pslot] = pltpu.async_copy(pbuf_, pdst, pssem)
        prev = (g, dst, slot)
    pg, pdst, pslot = prev
    pbuf_, _, pssem = slots[pslot]
    pg.wait()
    last_scatter[pslot] = pltpu.async_copy(pbuf_, pdst, pssem)
    for s in last_scatter.values():
        s.wait()


def _make_call():
    mesh = plsc.VectorSubcoreMesh(core_axis_name="c", subcore_axis_name="s",
                                  num_cores=NC, num_subcores=NS)
    return pl.kernel(
        _sc_body,
        out_type=(
            jax.ShapeDtypeStruct((2 * B, W_ROW), jnp.float32),
            jax.ShapeDtypeStruct((2 * B, SEQ_PAD), jnp.int32),
        ),
        mesh=mesh,
        scratch_types=[
            pltpu.VMEM((BPW,), jnp.int32),
            pltpu.VMEM((2 * BPW // CTX_CH, CTX_CH), jnp.int32),
            pltpu.VMEM((2 * BPW // SUF_CH, SUF_CH), jnp.int32),
            pltpu.VMEM((BPW, W_PRE), jnp.float32),
            pltpu.VMEM((BPW, SEQ_PAD), jnp.int32),
            pltpu.VMEM((CTX_CH, W_CTX), jnp.float32),
            pltpu.VMEM((CTX_CH, W_CTX), jnp.float32),
            pltpu.VMEM((SUF_CH, W_SUF), jnp.float32),
            pltpu.VMEM((SUF_CH, W_SUF), jnp.float32),
        ] + [pltpu.SemaphoreType.DMA] * 12,
        compiler_params=pltpu.CompilerParams(use_tc_tiling_on_sc=False),
    )


@jax.jit
def _prompt_gather(cls_id, ctx_pos2, ctx_neg2, pre_pos2, suf_pos2,
                   pre_neg2, suf_neg2, tok_neg, tok_pos):
    cls_w = cls_id.reshape(NW, BPW)
    cls2 = jnp.concatenate([cls_id.reshape(NW, BPW)] * 2, axis=1)
    cls_c = cls2.reshape(NW, 2 * BPW // CTX_CH, CTX_CH)
    cls_s = cls2.reshape(NW, 2 * BPW // SUF_CH, SUF_CH)
    call = _make_call()
    return call(cls_w, cls_c, cls_s,
                pre_neg2, ctx_neg2, suf_neg2,
                pre_pos2, ctx_pos2, suf_pos2,
                tok_neg, tok_pos)


def kernel(cls_id, ctx_pos, ctx_neg, token_prefix_pos, token_suffix_pos,
           token_prefix_neg, token_suffix_neg, tokenized_prompts):
    n_cls = ctx_pos.shape[0]
    prompts2, tokenized = _prompt_gather(
        cls_id,
        ctx_pos.reshape(n_cls, W_CTX),
        ctx_neg.reshape(n_cls, W_CTX),
        token_prefix_pos.reshape(n_cls, W_PRE),
        token_suffix_pos.reshape(n_cls, W_SUF),
        token_prefix_neg.reshape(n_cls, W_PRE),
        token_suffix_neg.reshape(n_cls, W_SUF),
        jnp.pad(tokenized_prompts[:n_cls], ((0, 0), (0, SEQ_PAD - SEQ))),
        jnp.pad(tokenized_prompts[n_cls:], ((0, 0), (0, SEQ_PAD - SEQ))),
    )
    return prompts2.reshape(2 * B, SEQ, D), tokenized[:, :SEQ]


# speedup vs baseline: 1.6961x; 1.6806x over previous
import jax
import jax.numpy as jnp
from jax import lax
from jax.experimental import pallas as pl
from jax.experimental.pallas import tpu as pltpu
from jax.experimental.pallas import tpu_sc as plsc

N_CLS = 1000
N_CTX = 16
D = 512
SEQ = 77
SUF = SEQ - 1 - N_CTX
SUF_MAIN = 56
SUF_TAIL = 8
SUF_PAD = SUF_MAIN + SUF_TAIL
B = 1024
SEQ_PAD = 128

NC, NS = 2, 16
NW = NC * NS
BPW = B // NW
CTX_CH = 2
TAIL_CH = 4


def _sc_body(cls_w, cls_p, cls_c, cls_t,
             pre_n, ctx_n, suf_n, pre_p, ctx_p, suf_p,
             tail_n, tail_p, tok_n, tok_p,
             out_pre, out_ctx, out_suf, out_tok,
             idx_v, idxp_v, idxc_v, idxt_v,
             pbuf, tbuf, cbuf0, cbuf1, sbuf0, sbuf1, lbuf,
             gsem_p, ssem_p, gsem_t, ssem_t, gsem_l, ssem_l,
             gsem_c0, gsem_c1, ssem_c0, ssem_c1,
             gsem_s0, gsem_s1, ssem_s0, ssem_s1):
    wid = lax.axis_index("s") * NC + lax.axis_index("c")
    base = wid * BPW

    pltpu.sync_copy(cls_w.at[wid], idx_v)
    pltpu.sync_copy(cls_p.at[wid], idxp_v)
    pltpu.sync_copy(cls_c.at[wid], idxc_v)
    pltpu.sync_copy(cls_t.at[wid], idxt_v)

    slots = {
        "s0": (sbuf0, gsem_s0, ssem_s0),
        "s1": (sbuf1, gsem_s1, ssem_s1),
        "c0": (cbuf0, gsem_c0, ssem_c0),
        "c1": (cbuf1, gsem_c1, ssem_c1),
        "p": (pbuf, gsem_p, ssem_p),
        "t": (tbuf, gsem_t, ssem_t),
        "l": (lbuf, gsem_l, ssem_l),
    }

    task_list = []
    halves = ((pre_n, ctx_n, suf_n, tail_n, tok_n, 0),
              (pre_p, ctx_p, suf_p, tail_p, tok_p, 1))
    for pol, (pre_t, ctx_t, suf_t, tail_t, tok_t, _) in enumerate(halves):
        ob = pol * B + base
        for j in range(BPW):
            idx = idxc_v.at[pol * (BPW // CTX_CH) + j // CTX_CH,
                            pl.ds(j % CTX_CH, 1)]
            dst = out_suf.at[pl.ds(ob + j, 1), pl.ds(0, SUF_MAIN), :]
            task_list.append((suf_t.at[idx, pl.ds(0, SUF_MAIN), :], dst,
                              "s0" if j % 2 == 0 else "s1"))
            if j % CTX_CH == 0:
                jc = j // CTX_CH
                idxc = idxc_v.at[pol * (BPW // CTX_CH) + jc]
                cdst = out_ctx.at[pl.ds(ob + jc * CTX_CH, CTX_CH)]
                task_list.append((ctx_t.at[idxc], cdst,
                                  "c0" if jc % 2 == 0 else "c1"))
            if j % TAIL_CH == 0:
                jt = j // TAIL_CH
                idxt = idxt_v.at[pol * (BPW // TAIL_CH) + jt]
                ldst = out_suf.at[pl.ds(ob + jt * TAIL_CH, TAIL_CH),
                                  pl.ds(SUF_MAIN, SUF_TAIL), :]
                task_list.append((tail_t.at[idxt], ldst, "l"))
            if j in (1, 17):
                jp = j // 16
                task_list.append((pre_t.at[idxp_v.at[jp]],
                                  out_pre.at[pl.ds(ob + jp * (BPW // 2),
                                                   BPW // 2)], "p"))
            if j in (3, 19):
                jp = j // 16
                task_list.append((tok_t.at[idxp_v.at[jp]],
                                  out_tok.at[pl.ds(ob + jp * (BPW // 2),
                                                   BPW // 2), :], "t"))

    last_scatter = {}
    prev = None
    for src, dst, slot in task_list:
        buf, gsem, _ = slots[slot]
        if slot in last_scatter:
            last_scatter.pop(slot).wait()
        g = pltpu.async_copy(src, buf, gsem)
        if prev is not None:
            pg, pdst, pslot = prev
            pbuf_, _, pssem = slots[pslot]
            pg.wait()
            last_scatter[pslot] = pltpu.async_copy(pbuf_, pdst, pssem)
        prev = (g, dst, slot)
    pg, pdst, pslot = prev
    pbuf_, _, pssem = slots[pslot]
    pg.wait()
    last_scatter[pslot] = pltpu.async_copy(pbuf_, pdst, pssem)
    for s in last_scatter.values():
        s.wait()


def _make_call():
    mesh = plsc.VectorSubcoreMesh(core_axis_name="c", subcore_axis_name="s",
                                  num_cores=NC, num_subcores=NS)
    return pl.kernel(
        _sc_body,
        out_type=(
            jax.ShapeDtypeStruct((2 * B, D), jnp.float32),
            jax.ShapeDtypeStruct((2 * B, N_CTX, D), jnp.float32),
            jax.ShapeDtypeStruct((2 * B, SUF_PAD, D), jnp.float32),
            jax.ShapeDtypeStruct((2 * B, SEQ_PAD), jnp.int32),
        ),
        mesh=mesh,
        scratch_types=[
            pltpu.VMEM((BPW,), jnp.int32),
            pltpu.VMEM((2, BPW // 2), jnp.int32),
            pltpu.VMEM((2 * BPW // CTX_CH, CTX_CH), jnp.int32),
            pltpu.VMEM((2 * BPW // TAIL_CH, TAIL_CH), jnp.int32),
            pltpu.VMEM((BPW // 2, D), jnp.float32),
            pltpu.VMEM((BPW // 2, SEQ_PAD), jnp.int32),
            pltpu.VMEM((CTX_CH, N_CTX, D), jnp.float32),
            pltpu.VMEM((CTX_CH, N_CTX, D), jnp.float32),
            pltpu.VMEM((1, SUF_MAIN, D), jnp.float32),
            pltpu.VMEM((1, SUF_MAIN, D), jnp.float32),
            pltpu.VMEM((TAIL_CH, SUF_TAIL, D), jnp.float32),
        ] + [pltpu.SemaphoreType.DMA] * 14,
    )


@jax.jit
def _prompt_gather(cls_id, ctx_pos, ctx_neg, pre_pos2, suf_pos,
                   pre_neg2, suf_neg, tail_pos, tail_neg, tok_neg, tok_pos):
    cls_w = cls_id.reshape(NW, BPW)
    cls2 = jnp.concatenate([cls_w] * 2, axis=1)
    cls_p = cls_w.reshape(NW, 2, BPW // 2)
    cls_c = cls2.reshape(NW, 2 * BPW // CTX_CH, CTX_CH)
    cls_t = cls2.reshape(NW, 2 * BPW // TAIL_CH, TAIL_CH)
    call = _make_call()
    pre, ctx, suf, tok = call(cls_w, cls_p, cls_c, cls_t,
                              pre_neg2, ctx_neg, suf_neg,
                              pre_pos2, ctx_pos, suf_pos,
                              tail_neg, tail_pos, tok_neg, tok_pos)
    prompts = jnp.concatenate(
        [pre.reshape(2 * B, 1, D), ctx, suf[:, :SUF, :]], axis=1)
    return prompts, tok[:, :SEQ]


def kernel(cls_id, ctx_pos, ctx_neg, token_prefix_pos, token_suffix_pos,
           token_prefix_neg, token_suffix_neg, tokenized_prompts):
    n_cls = ctx_pos.shape[0]
    pad_tail = ((0, 0), (0, SUF_TAIL - (SUF - SUF_MAIN)), (0, 0))
    return _prompt_gather(
        cls_id, ctx_pos, ctx_neg,
        token_prefix_pos.reshape(n_cls, D),
        token_suffix_pos,
        token_prefix_neg.reshape(n_cls, D),
        token_suffix_neg,
        jnp.pad(token_suffix_pos[:, SUF_MAIN:, :], pad_tail),
        jnp.pad(token_suffix_neg[:, SUF_MAIN:, :], pad_tail),
        jnp.pad(tokenized_prompts[:n_cls], ((0, 0), (0, SEQ_PAD - SEQ))),
        jnp.pad(tokenized_prompts[n_cls:], ((0, 0), (0, SEQ_PAD - SEQ))),
    )
